# Initial kernel scaffold; baseline (speedup 1.0000x reference)
import jax
import jax.numpy as jnp
from jax.experimental import pallas as pl


def kernel(embs, edge_index, rel_type, batch_size, weight1, root1, bias1, weight2, root2, bias2):
    raise NotImplementedError("write your pallas kernel here")


# speedup vs baseline: 5.9905x; 5.9905x over previous
import functools

import jax
import jax.numpy as jnp
from jax import lax
from jax.experimental import pallas as pl
from jax.experimental.pallas import tpu as pltpu
from jax.experimental.pallas import tpu_sc as plsc

N = 10000
E = 320000
D = 128
R = 8

DA = D + 16
CW = 48
NCC = DA // CW

NSC = 2
NT = 16
RPC = R // NSC
K = 128
NB = 157
EPT = NB * K
EPAD = NT * EPT

ACC_R = RPC * N + 16
TRASH = RPC * N
ZPT = ACC_R // NT
CRT = RPC * N // NT


def _sidx_body(dst_ref, rel_ref, o_ref):
    dv = dst_ref[...]
    rv = rel_ref[...]
    for c in range(NSC):
        rlv = rv - c * RPC
        valid = (rlv >= 0) & (rlv < RPC)
        o_ref[c] = jnp.where(valid, rlv * N + dv, TRASH)


def _build_sidx(dst_p, rel_p):
    return pl.pallas_call(
        _sidx_body,
        out_shape=jax.ShapeDtypeStruct((NSC, NT, NB, K), jnp.int32),
    )(dst_p, rel_p)


def _sc_agg_body(xa_hbm, xb_hbm, xc_hbm, src_hbm, sidx_hbm, zeros_hbm,
                 agg_hbm, srcb, sidxb, rows, acc, sem):
    c = lax.axis_index("c")
    s = lax.axis_index("s")

    for cc in range(NCC):
        xin = (xa_hbm, xb_hbm, xc_hbm)[cc]

        pltpu.sync_copy(zeros_hbm, acc.at[pl.ds(s * ZPT, ZPT)])
        plsc.subcore_barrier()

        def batch_body(b, _):
            pltpu.sync_copy(src_hbm.at[s, b], srcb)
            pltpu.sync_copy(sidx_hbm.at[c, s, b], sidxb)
            pltpu.async_copy(xin.at[srcb], rows, sem).wait()
            pltpu.sync_copy(rows, acc.at[sidxb], add=True)
            return 0
        lax.fori_loop(0, NB, batch_body, 0)
        plsc.subcore_barrier()

        rloc = s // 4
        n0 = (s % 4) * CRT
        pltpu.sync_copy(acc.at[pl.ds(s * CRT, CRT)],
                        agg_hbm.at[cc, c * RPC + rloc, pl.ds(n0, CRT)])
        plsc.subcore_barrier()


_sc_agg = pl.kernel(
    _sc_agg_body,
    out_type=jax.ShapeDtypeStruct((NCC, R, N, CW), jnp.float32),
    mesh=plsc.VectorSubcoreMesh(
        core_axis_name="c", subcore_axis_name="s",
        num_cores=NSC, num_subcores=NT),
    scratch_types=[
        pltpu.VMEM((K,), jnp.int32),
        pltpu.VMEM((K,), jnp.int32),
        pltpu.VMEM((K, CW), jnp.float32),
        pltpu.VMEM_SHARED((ACC_R, CW), jnp.float32),
        pltpu.SemaphoreType.DMA,
    ],
    compiler_params=pltpu.CompilerParams(use_tc_tiling_on_sc=False),
)


def _tc_layer_body(apply_gelu, xa_ref, xb_ref, xc_ref, agg_ref, w_ref,
                   root_ref, bias_ref, *out_refs):
    xfull = jnp.concatenate(
        [xa_ref[...], xb_ref[...], xc_ref[...]], axis=1)
    acc = jnp.dot(xfull[:, :D], root_ref[...],
                  preferred_element_type=jnp.float32)
    for r in range(R):
        a = jnp.concatenate(
            [agg_ref[0, r], agg_ref[1, r], agg_ref[2, r]], axis=1)
        scale = 1.0 / jnp.maximum(a[:, D:D + 1], 1.0)
        acc += jnp.dot(a[:, :D] * scale, w_ref[r],
                       preferred_element_type=jnp.float32)
    acc = acc + bias_ref[...]
    if apply_gelu:
        acc = acc * 0.5 * (1.0 + lax.erf(acc * (2.0 ** -0.5)))
        y = jnp.concatenate(
            [acc, jnp.ones((acc.shape[0], 1), jnp.float32),
             jnp.zeros((acc.shape[0], DA - D - 1), jnp.float32)], axis=1)
        out_refs[0][...] = y[:, :CW]
        out_refs[1][...] = y[:, CW:2 * CW]
        out_refs[2][...] = y[:, 2 * CW:]
    else:
        out_refs[0][...] = acc


def _tc_layer(xa, xb, xc, agg, weight, root, bias, apply_gelu):
    BN = 400
    if apply_gelu:
        out_shape = [jax.ShapeDtypeStruct((N, CW), jnp.float32)] * 3
        out_specs = [pl.BlockSpec((BN, CW), lambda i: (i, 0))] * 3
    else:
        out_shape = jax.ShapeDtypeStruct((N, D), jnp.float32)
        out_specs = pl.BlockSpec((BN, D), lambda i: (i, 0))
    return pl.pallas_call(
        functools.partial(_tc_layer_body, apply_gelu),
        grid=(N // BN,),
        in_specs=[
            pl.BlockSpec((BN, CW), lambda i: (i, 0)),
            pl.BlockSpec((BN, CW), lambda i: (i, 0)),
            pl.BlockSpec((BN, CW), lambda i: (i, 0)),
            pl.BlockSpec((NCC, R, BN, CW), lambda i: (0, 0, i, 0)),
            pl.BlockSpec((R, D, D), lambda i: (0, 0, 0)),
            pl.BlockSpec((D, D), lambda i: (0, 0)),
            pl.BlockSpec((1, D), lambda i: (0, 0)),
        ],
        out_specs=out_specs,
        out_shape=out_shape,
    )(xa, xb, xc, agg, weight, root, bias)


def kernel(embs, edge_index, rel_type, batch_size, weight1, root1, bias1,
           weight2, root2, bias2):
    src = edge_index[0]
    dst = edge_index[1]
    pad = EPAD - E
    src_p = jnp.concatenate(
        [src, jnp.zeros((pad,), jnp.int32)]).reshape(NT, NB, K)
    dst_p = jnp.concatenate(
        [dst, jnp.zeros((pad,), jnp.int32)]).reshape(NT, NB, K)
    rel_p = jnp.concatenate(
        [rel_type, jnp.full((pad,), R, jnp.int32)]).reshape(NT, NB, K)
    sidx = _build_sidx(dst_p, rel_p)
    zeros_acc = jnp.zeros((ZPT, CW), jnp.float32)

    xa = embs[:, :CW]
    xb = embs[:, CW:2 * CW]
    xc = jnp.concatenate(
        [embs[:, 2 * CW:], jnp.ones((N, 1), jnp.float32),
         jnp.zeros((N, DA - D - 1), jnp.float32)], axis=1)

    agg1 = _sc_agg(xa, xb, xc, src_p, sidx, zeros_acc)
    xa1, xb1, xc1 = _tc_layer(xa, xb, xc, agg1, weight1, root1,
                              bias1.reshape(1, D), True)
    agg2 = _sc_agg(xa1, xb1, xc1, src_p, sidx, zeros_acc)
    out = _tc_layer(xa1, xb1, xc1, agg2, weight2, root2,
                    bias2.reshape(1, D), False)
    return out
